# Initial kernel scaffold; baseline (speedup 1.0000x reference)
import jax
import jax.numpy as jnp
from jax.experimental import pallas as pl


def kernel(classifications, bbox_regressions, ldm_regressions, anchors, annotations):
    raise NotImplementedError("write your pallas kernel here")


# speedup vs baseline: 2.4766x; 2.4766x over previous
import functools

import jax
import jax.numpy as jnp
import numpy as np
from jax.experimental import pallas as pl
from jax.experimental.pallas import tpu as pltpu

_OMEGA = 3.0
_EPSILON = 2.0
_WING_C = 3.0 - 3.0 * float(np.log1p(_OMEGA / _EPSILON))

_INT_MIN = np.int32(-(2 ** 31))
_INT_MAX = np.int32(2 ** 31 - 1)
_XOR = np.int32(0x7FFFFFFF)


def _keys_from(vals):
    bits = jax.lax.bitcast_convert_type(vals, jnp.int32)
    return jnp.where(bits < 0, jnp.bitwise_xor(bits, _XOR), bits)


def _vals_from(keys):
    return jax.lax.bitcast_convert_type(
        jnp.where(keys < 0, jnp.bitwise_xor(keys, _XOR), keys), jnp.float32)


def _loss_body(ann_t_ref, ann_ref, anchors_ref, cls_ref, bbox_ref, ldm_ref,
               out_ref, keys_scr, acc, *, nb, n_ann, batch):
    j = pl.program_id(0)
    i = pl.program_id(1)

    @pl.when(jnp.logical_and(j == 0, i == 0))
    def _():
        acc[6] = 0.0
        acc[7] = 0.0
        acc[8] = 0.0

    @pl.when(i == 0)
    def _():
        acc[0] = 0.0
        acc[1] = 0.0
        acc[2] = 0.0
        acc[3] = 0.0
        acc[4] = 0.0
        acc[5] = 0.0

    ann_t = ann_t_ref[0]
    ann = ann_ref[0]
    anchors = anchors_ref[0]
    cls = cls_ref[0]
    bbox_reg = bbox_ref[0]
    ldm_reg = ldm_ref[0]

    a0 = anchors[:, 0:1]
    a1 = anchors[:, 1:2]
    a2 = anchors[:, 2:3]
    a3 = anchors[:, 3:4]
    b0 = ann_t[0:1, :]
    b1 = ann_t[1:2, :]
    b2 = ann_t[2:3, :]
    b3 = ann_t[3:4, :]
    valid = b0 > 0.0

    iw = jnp.maximum(jnp.minimum(a2, b2) - jnp.maximum(a0, b0), 0.0)
    ih = jnp.maximum(jnp.minimum(a3, b3) - jnp.maximum(a1, b1), 0.0)
    inter = iw * ih
    ua = jnp.maximum((a2 - a0) * (a3 - a1) + (b2 - b0) * (b3 - b1) - inter,
                     1e-08)
    iou = jnp.where(valid, inter / ua, -1.0)
    iou_max = jnp.max(iou, axis=1, keepdims=True)
    cols = jax.lax.broadcasted_iota(jnp.int32, iou.shape, 1)
    idx = jnp.min(jnp.where(iou == iou_max, cols, n_ann), axis=1,
                  keepdims=True)
    onehot = cols == idx

    pos = iou_max >= 0.7
    neg = iou_max < 0.4
    posf = pos.astype(jnp.float32)
    acc[0] = acc[0] + jnp.sum(posf)
    acc[1] = acc[1] + jnp.sum(neg.astype(jnp.float32))
    acc[2] = acc[2] + jnp.sum(-cls[:, 0:1] * posf)

    neg_vals = jnp.where(neg, -cls[:, 1:2], -jnp.inf)
    keys = _keys_from(neg_vals)
    lane = jax.lax.broadcasted_iota(jnp.int32, keys_scr.shape, 1)
    keys_scr[:, :] = jnp.where(lane == i, keys, keys_scr[:, :])

    aw = a2 - a0
    ah = a3 - a1
    acx = a0 + 0.5 * aw
    acy = a1 + 0.5 * ah
    g0 = jnp.sum(jnp.where(onehot, b0, 0.0), axis=1, keepdims=True)
    g1 = jnp.sum(jnp.where(onehot, b1, 0.0), axis=1, keepdims=True)
    g2 = jnp.sum(jnp.where(onehot, b2, 0.0), axis=1, keepdims=True)
    g3 = jnp.sum(jnp.where(onehot, b3, 0.0), axis=1, keepdims=True)
    gw = g2 - g0
    gh = g3 - g1
    gcx = g0 + 0.5 * gw
    gcy = g1 + 0.5 * gh
    tdx = (gcx - acx) / (aw + 1e-14)
    tdy = (gcy - acy) / (ah + 1e-14)
    tdw = jnp.log(jnp.where(pos, gw, aw) / aw)
    tdh = jnp.log(jnp.where(pos, gh, ah) / ah)
    bt = jnp.concatenate([tdx / 0.1, tdy / 0.1, tdw / 0.2, tdh / 0.2],
                         axis=1)
    diff = jnp.abs(bt - bbox_reg)
    sl1 = jnp.where(diff < 1.0, 0.5 * diff * diff, diff - 0.5)
    acc[3] = acc[3] + jnp.sum(sl1 * posf)

    ldm_ann = ann[:, 4:]
    assigned = jnp.dot(onehot.astype(jnp.float32), ldm_ann,
                       preferred_element_type=jnp.float32)
    rs = jnp.sum(assigned, axis=1, keepdims=True)
    lposf = jnp.where(jnp.logical_and(rs > 0.0, pos), 1.0, 0.0)
    acc[5] = acc[5] + jnp.sum(lposf)
    isx = (jax.lax.broadcasted_iota(jnp.int32, (1, 196), 1) % 2) == 0
    denom = jnp.where(isx, aw, ah) + 1e-14
    ctr = jnp.where(isx, acx, acy)
    lt = (assigned - ctr) / denom / 0.1
    delta = jnp.abs(lt - ldm_reg)
    wl = jnp.where(delta < _OMEGA, _OMEGA * jnp.log1p(delta / _EPSILON),
                   delta - _WING_C)
    acc[4] = acc[4] + jnp.sum(wl * lposf)

    @pl.when(i == nb - 1)
    def _():
        num_pos = acc[0]
        num_neg = acc[1]
        count_f = jnp.minimum(num_pos * 3.0, num_neg)
        count = count_f.astype(jnp.int32)
        keys_all = keys_scr[:, :]

        def step(_, lohi):
            lo, hi = lohi
            mid = (lo >> 1) + (hi >> 1) + ((lo | hi) & 1)
            c_ge = jnp.sum(jnp.where(keys_all >= mid, 1, 0))
            ok = c_ge >= count
            lo2 = jnp.where(ok, mid, lo)
            hi2 = jnp.where(ok, hi, mid - 1)
            cont = lo < hi
            return (jnp.where(cont, lo2, lo), jnp.where(cont, hi2, hi))

        kth, _ = jax.lax.fori_loop(0, 33, step, (_INT_MIN, _INT_MAX))
        vals_all = _vals_from(keys_all)
        gt = keys_all > kth
        c_gt = jnp.sum(jnp.where(gt, 1.0, 0.0))
        sum_gt = jnp.sum(jnp.where(gt, vals_all, 0.0))
        val_k = _vals_from(kth)
        neg_sum = sum_gt + (count_f - c_gt) * val_k
        neg_mean = jnp.where(count_f > 0.0,
                             neg_sum / jnp.maximum(count_f, 1.0), 0.0)

        has_ann = jnp.max(valid.astype(jnp.float32))
        pos_mean = acc[2] / jnp.maximum(num_pos, 1.0)
        cls_l = jnp.where(num_pos > 0.0, pos_mean + neg_mean, 0.0) * has_ann
        box_l = jnp.where(num_pos > 0.0,
                          acc[3] / jnp.maximum(num_pos * 4.0, 1.0),
                          0.0) * has_ann
        ldm_l = jnp.where(acc[5] > 0.0,
                          acc[4] / jnp.maximum(acc[5] * 196.0, 1.0),
                          0.0) * has_ann
        acc[6] = acc[6] + cls_l / batch
        acc[7] = acc[7] + box_l / batch
        acc[8] = acc[8] + ldm_l / batch
        out_ref[:, :] = jnp.concatenate(
            [jnp.broadcast_to(acc[6], (1, 1)),
             jnp.broadcast_to(acc[7], (1, 1)),
             jnp.broadcast_to(acc[8], (1, 1))], axis=1)


def kernel(classifications, bbox_regressions, ldm_regressions, anchors,
           annotations):
    B, A, _ = classifications.shape
    N = annotations.shape[1]
    ba = A
    for cand in (1000, 1024, 800, 640, 500, 400, 256, 200, 160, 128, 125):
        if A % cand == 0:
            ba = cand
            break
    nb = A // ba
    ann_t = jnp.transpose(annotations[:, :, :4], (0, 2, 1))

    body = functools.partial(_loss_body, nb=nb, n_ann=N, batch=float(B))
    out = pl.pallas_call(
        body,
        grid=(B, nb),
        in_specs=[
            pl.BlockSpec((1, 4, N), lambda j, i: (j, 0, 0)),
            pl.BlockSpec((1, N, 200), lambda j, i: (j, 0, 0)),
            pl.BlockSpec((1, ba, 4), lambda j, i: (0, i, 0)),
            pl.BlockSpec((1, ba, 2), lambda j, i: (j, i, 0)),
            pl.BlockSpec((1, ba, 4), lambda j, i: (j, i, 0)),
            pl.BlockSpec((1, ba, 196), lambda j, i: (j, i, 0)),
        ],
        out_specs=pl.BlockSpec((1, 3), lambda j, i: (0, 0)),
        out_shape=jax.ShapeDtypeStruct((1, 3), jnp.float32),
        scratch_shapes=[
            pltpu.VMEM((ba, nb), jnp.int32),
            pltpu.SMEM((16,), jnp.float32),
        ],
        compiler_params=pltpu.CompilerParams(
            dimension_semantics=("arbitrary", "arbitrary")),
    )(ann_t, annotations, anchors, classifications, bbox_regressions,
      ldm_regressions)
    return out[0]
